# baseline (speedup 1.0000x reference)
import functools

import jax
import jax.numpy as jnp
from jax import lax
from jax.experimental import pallas as pl
from jax.experimental.pallas import tpu as pltpu
from jax.experimental.pallas import tpu_sc as plsc

N = 10000
E = 320000
D = 128
H = 128
C = 8
G = 64

TILES = 32
DCH = 80
K = 128
EPAD = TILES * DCH * K
EPT = DCH * K
NP = 10240
NH = NP // 2
RPT = NP // 16
RPT2 = NH // 16
CAPT2 = EPT + 256
IB = 4

_mesh = plsc.VectorSubcoreMesh(core_axis_name="c", subcore_axis_name="s")


@functools.partial(
    pl.kernel,
    mesh=_mesh,
    out_type=jax.ShapeDtypeStruct((2, NP), jnp.float32),
    scratch_types=[
        pltpu.VMEM((DCH, K), jnp.int32),
        pltpu.VMEM((DCH, K), jnp.float32),
        pltpu.VMEM_SHARED((NP,), jnp.float32),
        pltpu.SemaphoreType.DMA,
    ],
)
def _deg_sc(dst_hbm, ew_hbm, z1_hbm, out_hbm, dst_v, ew_v, acc, sd):
    c = lax.axis_index("c")
    s = lax.axis_index("s")
    b = c * 16 + s
    r0 = pl.multiple_of(s * RPT, 8)
    pltpu.sync_copy(z1_hbm, acc.at[pl.ds(r0, RPT)])
    plsc.subcore_barrier()
    pltpu.sync_copy(dst_hbm.at[b], dst_v)
    pltpu.sync_copy(ew_hbm.at[b], ew_v)

    win = 8

    def chunk(j, carry):
        pltpu.async_copy(ew_v.at[j], acc.at[dst_v.at[j]], sd, add=True)

        @pl.when(j >= win)
        def _():
            pltpu.make_async_copy(ew_v.at[j - win],
                                  acc.at[dst_v.at[j - win]], sd).wait()

        return carry

    lax.fori_loop(0, DCH, chunk, 0)

    def drain(j, carry):
        pltpu.make_async_copy(ew_v.at[j], acc.at[dst_v.at[j]], sd).wait()
        return carry

    lax.fori_loop(DCH - win, DCH, drain, 0)
    plsc.subcore_barrier()
    pltpu.sync_copy(acc.at[pl.ds(r0, RPT)], out_hbm.at[c, pl.ds(r0, RPT)])


@functools.partial(
    pl.kernel,
    mesh=_mesh,
    out_type=[
        jax.ShapeDtypeStruct((4 * TILES * CAPT2,), jnp.int32),
        jax.ShapeDtypeStruct((4 * TILES * CAPT2,), jnp.int32),
        jax.ShapeDtypeStruct((4 * TILES * CAPT2,), jnp.float32),
        jax.ShapeDtypeStruct((TILES, 16), jnp.int32),
    ],
    scratch_types=[
        pltpu.VMEM((EPT,), jnp.int32),
        pltpu.VMEM((EPT,), jnp.int32),
        pltpu.VMEM((EPT,), jnp.float32),
        pltpu.VMEM((K,), jnp.int32),
        pltpu.VMEM((K,), jnp.int32),
        pltpu.VMEM((K,), jnp.int32),
        pltpu.VMEM((K,), jnp.int32),
        pltpu.VMEM((K,), jnp.float32),
        pltpu.VMEM((16,), jnp.int32),
        pltpu.SemaphoreType.DMA,
    ],
)
def _bkt_sc(src_hbm, dst_hbm, ew_hbm, bs_hbm, bd_hbm, bw_hbm, bc_hbm,
            sv, dv, wv, posb, posc, pb_s, pb_d, pb_w, cntv, sd):
    c = lax.axis_index("c")
    s = lax.axis_index("s")
    b = c * 16 + s
    pltpu.sync_copy(src_hbm.at[b], sv)
    pltpu.sync_copy(dst_hbm.at[b], dv)
    pltpu.sync_copy(ew_hbm.at[b], wv)

    lane = lax.iota(jnp.int32, 16)
    pads = [((q >> 1) * NH, (q & 1) * NH) for q in range(4)]
    gbs = [(q * TILES + b) * CAPT2 for q in range(4)]

    def _prefix(v):
        u = v
        for shf in (1, 2, 4, 8):
            shifted = u.at[jnp.maximum(lane - shf, 0)].get(
                mode="promise_in_bounds")
            u = u + jnp.where(lane >= shf, shifted, 0)
        return u

    def _flush(ch_start):
        @pl.when(ch_start >= K)
        def _():
            prev = pl.ds(ch_start - K, K)
            pltpu.make_async_copy(sv.at[prev], bs_hbm.at[posc], sd).wait()
            pltpu.make_async_copy(dv.at[prev], bd_hbm.at[posc], sd).wait()
            pltpu.make_async_copy(wv.at[prev], bw_hbm.at[posc], sd).wait()

        for g in range(K // 16):
            sl = pl.ds(g * 16, 16)
            posc[sl] = posb[sl]
        src_sl = pl.ds(ch_start, K)
        pltpu.async_copy(sv.at[src_sl], bs_hbm.at[posc], sd)
        pltpu.async_copy(dv.at[src_sl], bd_hbm.at[posc], sd)
        pltpu.async_copy(wv.at[src_sl], bw_hbm.at[posc], sd)

    def grp(t, carry):
        cur = carry
        base = t * 16
        s16 = sv[pl.ds(base, 16)]
        d16 = dv[pl.ds(base, 16)]
        sh = lax.shift_right_logical(s16 * 3277, 24)
        dh = lax.shift_right_logical(d16 * 3277, 24)
        bq = sh * 2 + dh
        excl = []
        kqs = []
        for q in range(4):
            v = jnp.where(bq == q, 1, 0)
            incl = _prefix(v)
            kqs.append(jnp.squeeze(lax.slice(incl, (15,), (16,))))
            excl.append(incl - v)
        base16 = jnp.where(
            bq == 0, gbs[0] + cur[0],
            jnp.where(bq == 1, gbs[1] + cur[1],
                      jnp.where(bq == 2, gbs[2] + cur[2],
                                gbs[3] + cur[3])))
        rank16 = jnp.where(
            bq == 0, excl[0],
            jnp.where(bq == 1, excl[1], jnp.where(bq == 2, excl[2], excl[3])))
        gg = t & 7
        posb[pl.ds(gg * 16, 16)] = base16 + rank16

        @pl.when(gg == 7)
        def _():
            _flush(t * 16 - 112)

        return tuple(cur[q] + kqs[q] for q in range(4))

    z = jnp.int32(0)
    fin = lax.fori_loop(0, EPT // 16, grp, (z,) * 4)

    last = pl.ds(EPT - K, K)
    pltpu.make_async_copy(sv.at[last], bs_hbm.at[posc], sd).wait()
    pltpu.make_async_copy(dv.at[last], bd_hbm.at[posc], sd).wait()
    pltpu.make_async_copy(wv.at[last], bw_hbm.at[posc], sd).wait()

    for q in range(4):
        for g in range(K // 16):
            sl = pl.ds(g * 16, 16)
            posb[sl] = gbs[q] + fin[q] + g * 16 + lane
            pb_s[sl] = jnp.full((16,), pads[q][0], jnp.int32)
            pb_d[sl] = jnp.full((16,), pads[q][1], jnp.int32)
            pb_w[sl] = jnp.zeros((16,), jnp.float32)
        pltpu.sync_copy(pb_s, bs_hbm.at[posb])
        pltpu.sync_copy(pb_d, bd_hbm.at[posb])
        pltpu.sync_copy(pb_w, bw_hbm.at[posb])

    cvec = jnp.zeros((16,), jnp.int32)
    for q in range(4):
        nch_q = lax.shift_right_logical(fin[q] + (K - 1), 7)
        cvec = jnp.where(lane == q, nch_q, cvec)
    cntv[...] = cvec
    pltpu.sync_copy(cntv, bc_hbm.at[b])


@functools.partial(
    pl.kernel,
    mesh=_mesh,
    out_type=jax.ShapeDtypeStruct((NP, H), jnp.float32),
    scratch_types=[
        pltpu.VMEM_SHARED((NH, H), jnp.float32),
        pltpu.VMEM_SHARED((NH, H), jnp.float32),
        pltpu.VMEM((IB * K,), jnp.int32),
        pltpu.VMEM((IB * K,), jnp.int32),
        pltpu.VMEM((IB * K,), jnp.float32),
        pltpu.VMEM((IB, K), jnp.int32),
        pltpu.VMEM((IB, K), jnp.int32),
        pltpu.VMEM((K, H), jnp.float32),
        pltpu.VMEM((K, H), jnp.float32),
        pltpu.VMEM((16,), jnp.int32),
        pltpu.SemaphoreType.DMA,
        pltpu.SemaphoreType.DMA,
    ],
)
def _agg_sc(hp_hbm, bs_hbm, bd_hbm, bw_hbm, bc_hbm, z2_hbm, out_hbm,
            table, acc, ibf_s, ibf_d, ibf_w, ib_s2, ib_d2, r0b, r1b, cntv,
            gs0, gs1):
    c = lax.axis_index("c")
    s = lax.axis_index("s")
    r0 = pl.multiple_of(s * RPT2, 8)
    pltpu.sync_copy(z2_hbm, acc.at[pl.ds(r0, RPT2)])

    rows = (r0b, r1b)
    gsem = (gs0, gs1)

    def _xform(i, abase, cbase):
        def tg(g, carry):
            sl = pl.ds(g * 16, 16)
            fsl = pl.ds(i * K + g * 16, 16)
            ib_s2[i, sl] = ibf_s[fsl] - abase
            ib_d2[i, sl] = ibf_d[fsl] - cbase
            return carry

        lax.fori_loop(0, K // 16, tg, 0)

    def _gissue(i, u):
        pltpu.async_copy(table.at[ib_s2.at[i]], rows[u], gsem[u])

    def _gwait(i, u):
        pltpu.make_async_copy(table.at[ib_s2.at[i]], rows[u],
                              gsem[u]).wait()

    def _scale(i, u):
        X = rows[u]

        def grp(g, carry):
            w16 = ibf_w[pl.ds(i * K + g * 16, 16)]
            for l in range(16):
                wsp = w16.at[jnp.full((16,), l, jnp.int32)].get(
                    mode="promise_in_bounds")
                e = g * 16 + l
                for cg in range(H // 16):
                    sl = pl.ds(cg * 16, 16)
                    X[e, sl] = X[e, sl] * wsp
            return carry

        lax.fori_loop(0, K // 16, grp, 0)

    for a in range(2):
        pltpu.sync_copy(hp_hbm.at[pl.ds(pl.multiple_of(a * NH + s * RPT2, 8),
                                        RPT2)],
                        table.at[pl.ds(r0, RPT2)])
        plsc.subcore_barrier()
        q = 2 * a + c
        abase = jnp.int32(a * NH)
        cbase = c * NH

        def prod(pi, pcarry):
            p = 2 * s + pi
            pltpu.sync_copy(bc_hbm.at[p], cntv)
            c16 = cntv[...]
            rot = c16.at[(lax.iota(jnp.int32, 16) + q) & 15].get(
                mode="promise_in_bounds")
            nch = jnp.squeeze(lax.slice(rot, (0,), (1,)))

            seg = (q * TILES + p) * CAPT2

            def blk(bi, carry):
                off = pl.multiple_of(seg + bi * (IB * K), 8)
                pltpu.sync_copy(bs_hbm.at[pl.ds(off, IB * K)], ibf_s)
                pltpu.sync_copy(bd_hbm.at[pl.ds(off, IB * K)], ibf_d)
                pltpu.sync_copy(bw_hbm.at[pl.ds(off, IB * K)], ibf_w)
                ch0 = bi * IB
                _xform(0, abase, cbase)

                @pl.when(ch0 < nch)
                def _():
                    _gissue(0, 0)

                for i in range(IB):
                    u = i % 2
                    ch = ch0 + i

                    @pl.when(ch < nch)
                    def _(i=i, u=u, ch=ch):
                        _gwait(i, u)
                        if i + 1 < IB:
                            _xform(i + 1, abase, cbase)

                            @pl.when(ch + 1 < nch)
                            def _(i=i, u=u):
                                _gissue(i + 1, 1 - u)

                        _scale(i, u)
                        pltpu.sync_copy(rows[u], acc.at[ib_d2.at[i]],
                                        add=True)

                return carry

            nblk = lax.shift_right_logical(nch + (IB - 1), 2)
            lax.fori_loop(0, nblk, blk, 0)
            return pcarry

        lax.fori_loop(0, 2, prod, 0)
        plsc.subcore_barrier()
    pltpu.sync_copy(acc.at[pl.ds(r0, RPT2)],
                    out_hbm.at[pl.ds(pl.multiple_of(c * NH + s * RPT2, 8),
                                     RPT2)])


def _padrows(h):
    return jnp.concatenate([h, jnp.zeros((NP - N, H), jnp.float32)], axis=0)


def _tc1_body(deg0_ref, deg1_ref, x_ref, w_ref, dis_ref, hp_ref):
    deg = 1.0 + deg0_ref[...] + deg1_ref[...]
    dis = jnp.where(deg > 0, lax.rsqrt(deg), 0.0)
    dis_ref[...] = dis
    h = jnp.dot(x_ref[...], w_ref[...], preferred_element_type=jnp.float32,
                precision=lax.Precision.HIGHEST)
    hp_ref[...] = _padrows(h * dis)


def _tc_mid_body(a_ref, hp_ref, dis_ref, b_ref, w_ref, out_ref):
    dis = dis_ref[...]
    t = (a_ref[...] + hp_ref[...]) * dis + b_ref[...]
    o = jnp.maximum(t, 0.0)
    h = jnp.dot(o, w_ref[...], preferred_element_type=jnp.float32,
                precision=lax.Precision.HIGHEST)
    out_ref[...] = _padrows(h * dis)


def _tc_fin_body(a_ref, hp_ref, dis_ref, b_ref, brow_ref, wl_ref, bl_ref,
                 out_ref):
    o3 = (a_ref[...] + hp_ref[...]) * dis_ref[...] + b_ref[...]
    gid = lax.broadcasted_iota(jnp.int32, (G, N), 0)
    oh = (gid == brow_ref[...]).astype(jnp.float32)
    sums = jnp.dot(oh, o3, preferred_element_type=jnp.float32,
                   precision=lax.Precision.HIGHEST)
    cnt = jnp.dot(oh, jnp.ones((N, 1), jnp.float32),
                  preferred_element_type=jnp.float32,
                  precision=lax.Precision.HIGHEST)
    pooled = sums / jnp.maximum(cnt, 1.0)
    out_ref[...] = jnp.dot(pooled, wl_ref[...],
                           preferred_element_type=jnp.float32,
                           precision=lax.Precision.HIGHEST) + bl_ref[...]


def _pc(body, out_shapes):
    return pl.pallas_call(body, out_shape=out_shapes)


def _pad_to(a, n, dtype):
    return jnp.concatenate([a, jnp.zeros((n - a.shape[0],), dtype)])


def kernel(x, edge_index, edge_attr, batch, W1, b1, W2, b2, W3, b3, Wl, bl):
    srcp = _pad_to(edge_index[0], EPAD, jnp.int32)
    dstp = _pad_to(edge_index[1], EPAD, jnp.int32)
    ewp = _pad_to(edge_attr, EPAD, jnp.float32)
    dst3 = dstp.reshape(TILES, DCH, K)
    ew3 = ewp.reshape(TILES, DCH, K)
    srcF = srcp.reshape(TILES, EPT)
    dstF = dstp.reshape(TILES, EPT)
    ewF = ewp.reshape(TILES, EPT)
    z1 = jnp.zeros((RPT,), jnp.float32)
    z2 = jnp.zeros((RPT2, H), jnp.float32)
    brow = batch[None, :]

    bs, bd, bw, bc = _bkt_sc(srcF, dstF, ewF)
    deg2 = _deg_sc(dst3, ew3, z1)
    deg0 = deg2[0, :N, None]
    deg1 = deg2[1, :N, None]
    dis, hp1 = _pc(_tc1_body, [
        jax.ShapeDtypeStruct((N, 1), jnp.float32),
        jax.ShapeDtypeStruct((NP, H), jnp.float32),
    ])(deg0, deg1, x, W1)

    a1 = _agg_sc(hp1, bs, bd, bw, bc, z2)
    hp2 = _pc(_tc_mid_body, jax.ShapeDtypeStruct((NP, H), jnp.float32))(
        a1[:N], hp1[:N], dis, b1[None, :], W2)

    a2 = _agg_sc(hp2, bs, bd, bw, bc, z2)
    hp3 = _pc(_tc_mid_body, jax.ShapeDtypeStruct((NP, H), jnp.float32))(
        a2[:N], hp2[:N], dis, b2[None, :], W3)

    a3 = _agg_sc(hp3, bs, bd, bw, bc, z2)
    out = _pc(_tc_fin_body, jax.ShapeDtypeStruct((G, C), jnp.float32))(
        a3[:N], hp3[:N], dis, b3[None, :], brow, Wl, bl[None, :])
    return out


# speedup vs baseline: 2.2107x; 2.2107x over previous
import functools

import jax
import jax.numpy as jnp
from jax import lax
from jax.experimental import pallas as pl
from jax.experimental.pallas import tpu as pltpu
from jax.experimental.pallas import tpu_sc as plsc

N = 10000
E = 320000
D = 128
H = 128
C = 8
G = 64

TILES = 32
CHUNKS = 79
K = 128
EPAD = TILES * CHUNKS * K
NP = 10240
RPT = NP // 16

_mesh = plsc.VectorSubcoreMesh(core_axis_name="c", subcore_axis_name="s")


@functools.partial(
    pl.kernel,
    mesh=_mesh,
    out_type=jax.ShapeDtypeStruct((2, NP), jnp.float32),
    scratch_types=[
        pltpu.VMEM((CHUNKS, K), jnp.int32),
        pltpu.VMEM((CHUNKS, K), jnp.float32),
        pltpu.VMEM_SHARED((NP,), jnp.float32),
        pltpu.SemaphoreType.DMA,
    ],
)
def _deg_sc(dst_hbm, ew_hbm, z1_hbm, out_hbm, dst_v, ew_v, acc, sd):
    c = lax.axis_index("c")
    s = lax.axis_index("s")
    b = c * 16 + s
    r0 = pl.multiple_of(s * RPT, 8)
    pltpu.sync_copy(z1_hbm, acc.at[pl.ds(r0, RPT)])
    plsc.subcore_barrier()
    pltpu.sync_copy(dst_hbm.at[b], dst_v)
    pltpu.sync_copy(ew_hbm.at[b], ew_v)

    win = 8

    def chunk(j, carry):
        pltpu.async_copy(ew_v.at[j], acc.at[dst_v.at[j]], sd, add=True)

        @pl.when(j >= win)
        def _():
            pltpu.make_async_copy(ew_v.at[j - win],
                                  acc.at[dst_v.at[j - win]], sd).wait()

        return carry

    lax.fori_loop(0, CHUNKS, chunk, 0)

    def drain(j, carry):
        pltpu.make_async_copy(ew_v.at[j], acc.at[dst_v.at[j]], sd).wait()
        return carry

    lax.fori_loop(CHUNKS - win, CHUNKS, drain, 0)
    plsc.subcore_barrier()
    pltpu.sync_copy(acc.at[pl.ds(r0, RPT)], out_hbm.at[c, pl.ds(r0, RPT)])


@functools.partial(
    pl.kernel,
    mesh=_mesh,
    out_type=jax.ShapeDtypeStruct((2, NP, H), jnp.float32),
    scratch_types=[
        pltpu.VMEM((CHUNKS, K), jnp.int32),
        pltpu.VMEM((CHUNKS, K), jnp.int32),
        pltpu.VMEM((CHUNKS * K,), jnp.float32),
        pltpu.VMEM((K, H), jnp.float32),
        pltpu.VMEM_SHARED((NP, H), jnp.float32),
        pltpu.SemaphoreType.DMA,
    ],
)
def _agg_sc(hp_hbm, src_hbm, dst_hbm, ew_hbm, z2_hbm, out_hbm,
            src_v, dst_v, ew_v, rows_v, acc, sem):
    c = lax.axis_index("c")
    s = lax.axis_index("s")
    b = c * 16 + s
    r0 = pl.multiple_of(s * RPT, 8)
    pltpu.sync_copy(z2_hbm, acc.at[pl.ds(r0, RPT)])
    plsc.subcore_barrier()
    pltpu.sync_copy(src_hbm.at[b], src_v)
    pltpu.sync_copy(dst_hbm.at[b], dst_v)
    pltpu.sync_copy(ew_hbm.at[b], ew_v)

    def chunk(j, carry):
        pltpu.async_copy(hp_hbm.at[src_v.at[j]], rows_v, sem).wait()
        for g in range(8):
            w16 = ew_v[pl.ds(j * K + g * 16, 16)]
            for l in range(16):
                ws = w16.at[jnp.full((16,), l, jnp.int32)].get(
                    mode="promise_in_bounds")
                e = g * 16 + l
                for cg in range(8):
                    sl = pl.ds(cg * 16, 16)
                    rows_v[e, sl] = rows_v[e, sl] * ws
        pltpu.sync_copy(rows_v, acc.at[dst_v.at[j]], add=True)
        return carry

    lax.fori_loop(0, CHUNKS, chunk, 0)
    plsc.subcore_barrier()
    pltpu.sync_copy(acc.at[pl.ds(r0, RPT)], out_hbm.at[c, pl.ds(r0, RPT)])


def _tc1_body(deg0_ref, deg1_ref, x_ref, w_ref, dis_ref, hp_ref):
    deg = 1.0 + deg0_ref[...] + deg1_ref[...]
    dis = jnp.where(deg > 0, lax.rsqrt(deg), 0.0)
    dis_ref[...] = dis
    h = jnp.dot(x_ref[...], w_ref[...], preferred_element_type=jnp.float32,
                precision=lax.Precision.HIGHEST)
    hp_ref[...] = h * dis


def _tc_mid_body(a0_ref, a1_ref, hp_ref, dis_ref, b_ref, w_ref, out_ref):
    dis = dis_ref[...]
    t = (a0_ref[...] + a1_ref[...] + hp_ref[...]) * dis + b_ref[...]
    o = jnp.maximum(t, 0.0)
    out_ref[...] = jnp.dot(o, w_ref[...], preferred_element_type=jnp.float32,
                           precision=lax.Precision.HIGHEST) * dis


def _tc_fin_body(a0_ref, a1_ref, hp_ref, dis_ref, b_ref, brow_ref, wl_ref,
                 bl_ref, out_ref):
    o3 = (a0_ref[...] + a1_ref[...] + hp_ref[...]) * dis_ref[...] + b_ref[...]
    gid = lax.broadcasted_iota(jnp.int32, (G, N), 0)
    oh = (gid == brow_ref[...]).astype(jnp.float32)
    sums = jnp.dot(oh, o3, preferred_element_type=jnp.float32,
                   precision=lax.Precision.HIGHEST)
    cnt = jnp.dot(oh, jnp.ones((N, 1), jnp.float32),
                  preferred_element_type=jnp.float32,
                  precision=lax.Precision.HIGHEST)
    pooled = sums / jnp.maximum(cnt, 1.0)
    out_ref[...] = jnp.dot(pooled, wl_ref[...],
                           preferred_element_type=jnp.float32,
                           precision=lax.Precision.HIGHEST) + bl_ref[...]


def _pc(body, out_shapes):
    return pl.pallas_call(body, out_shape=out_shapes)


def _pad_to(a, n, dtype):
    return jnp.concatenate([a, jnp.zeros((n - a.shape[0],), dtype)])


def kernel(x, edge_index, edge_attr, batch, W1, b1, W2, b2, W3, b3, Wl, bl):
    src3 = _pad_to(edge_index[0], EPAD, jnp.int32).reshape(TILES, CHUNKS, K)
    dst3 = _pad_to(edge_index[1], EPAD, jnp.int32).reshape(TILES, CHUNKS, K)
    ew3 = _pad_to(edge_attr, EPAD, jnp.float32).reshape(TILES, CHUNKS, K)
    ew2 = ew3.reshape(TILES, CHUNKS * K)
    z1 = jnp.zeros((RPT,), jnp.float32)
    z2 = jnp.zeros((RPT, H), jnp.float32)
    brow = batch[None, :]

    deg2 = _deg_sc(dst3, ew3, z1)
    deg0 = deg2[0, :N, None]
    deg1 = deg2[1, :N, None]
    dis, hp1 = _pc(_tc1_body, [
        jax.ShapeDtypeStruct((N, 1), jnp.float32),
        jax.ShapeDtypeStruct((N, H), jnp.float32),
    ])(deg0, deg1, x, W1)

    a1 = _agg_sc(hp1, src3, dst3, ew2, z2)
    hp2 = _pc(_tc_mid_body, jax.ShapeDtypeStruct((N, H), jnp.float32))(
        a1[0, :N], a1[1, :N], hp1, dis, b1[None, :], W2)

    a2 = _agg_sc(hp2, src3, dst3, ew2, z2)
    hp3 = _pc(_tc_mid_body, jax.ShapeDtypeStruct((N, H), jnp.float32))(
        a2[0, :N], a2[1, :N], hp2, dis, b2[None, :], W3)

    a3 = _agg_sc(hp3, src3, dst3, ew2, z2)
    out = _pc(_tc_fin_body, jax.ShapeDtypeStruct((G, C), jnp.float32))(
        a3[0, :N], a3[1, :N], hp3, dis, b3[None, :], brow, Wl, bl[None, :])
    return out


# speedup vs baseline: 2.5270x; 1.1431x over previous
import functools

import jax
import jax.numpy as jnp
from jax import lax
from jax.experimental import pallas as pl
from jax.experimental.pallas import tpu as pltpu
from jax.experimental.pallas import tpu_sc as plsc

N = 10000
E = 320000
D = 128
H = 128
C = 8
G = 64

TILES = 32
CHUNKS = 79
K = 128
HK = K // 2
EPAD = TILES * CHUNKS * K
NP = 10240
RPT = NP // 16

_mesh = plsc.VectorSubcoreMesh(core_axis_name="c", subcore_axis_name="s")


@functools.partial(
    pl.kernel,
    mesh=_mesh,
    out_type=jax.ShapeDtypeStruct((2, NP), jnp.float32),
    scratch_types=[
        pltpu.VMEM((CHUNKS, K), jnp.int32),
        pltpu.VMEM((CHUNKS, K), jnp.float32),
        pltpu.VMEM_SHARED((NP,), jnp.float32),
        pltpu.SemaphoreType.DMA,
    ],
)
def _deg_sc(dst_hbm, ew_hbm, z1_hbm, out_hbm, dst_v, ew_v, acc, sd):
    c = lax.axis_index("c")
    s = lax.axis_index("s")
    b = c * 16 + s
    r0 = pl.multiple_of(s * RPT, 8)
    pltpu.sync_copy(z1_hbm, acc.at[pl.ds(r0, RPT)])
    plsc.subcore_barrier()
    pltpu.sync_copy(dst_hbm.at[b], dst_v)
    pltpu.sync_copy(ew_hbm.at[b], ew_v)

    win = 8

    def chunk(j, carry):
        pltpu.async_copy(ew_v.at[j], acc.at[dst_v.at[j]], sd, add=True)

        @pl.when(j >= win)
        def _():
            pltpu.make_async_copy(ew_v.at[j - win],
                                  acc.at[dst_v.at[j - win]], sd).wait()

        return carry

    lax.fori_loop(0, CHUNKS, chunk, 0)

    def drain(j, carry):
        pltpu.make_async_copy(ew_v.at[j], acc.at[dst_v.at[j]], sd).wait()
        return carry

    lax.fori_loop(CHUNKS - win, CHUNKS, drain, 0)
    plsc.subcore_barrier()
    pltpu.sync_copy(acc.at[pl.ds(r0, RPT)], out_hbm.at[c, pl.ds(r0, RPT)])


@functools.partial(
    pl.kernel,
    mesh=_mesh,
    out_type=jax.ShapeDtypeStruct((2, NP, H), jnp.float32),
    scratch_types=[
        pltpu.VMEM((CHUNKS, K), jnp.int32),
        pltpu.VMEM((CHUNKS, K), jnp.int32),
        pltpu.VMEM((CHUNKS * K,), jnp.float32),
        pltpu.VMEM((HK, H), jnp.float32),
        pltpu.VMEM((HK, H), jnp.float32),
        pltpu.VMEM_SHARED((NP, H), jnp.float32),
        pltpu.SemaphoreType.DMA,
        pltpu.SemaphoreType.DMA,
    ],
)
def _agg_sc(hp_hbm, src_hbm, dst_hbm, ew_hbm, z2_hbm, out_hbm,
            src_v, dst_v, ew_v, rows_a, rows_b, acc, sem_a, sem_b):
    c = lax.axis_index("c")
    s = lax.axis_index("s")
    b = c * 16 + s
    r0 = pl.multiple_of(s * RPT, 8)
    pltpu.sync_copy(z2_hbm, acc.at[pl.ds(r0, RPT)])
    plsc.subcore_barrier()
    pltpu.sync_copy(src_hbm.at[b], src_v)
    pltpu.sync_copy(dst_hbm.at[b], dst_v)
    pltpu.sync_copy(ew_hbm.at[b], ew_v)

    def _sidx(j, h):
        return src_v.at[j, pl.ds(h * HK, HK)]

    def _didx(j, h):
        return dst_v.at[j, pl.ds(h * HK, HK)]

    def section(j, h, cur, nxt, csem, nsem, last):
        pltpu.make_async_copy(hp_hbm.at[_sidx(j, h)], cur, csem).wait()
        if h == 0:
            pltpu.async_copy(hp_hbm.at[_sidx(j, 1)], nxt, nsem)
        elif not last:
            @pl.when(j + 1 < CHUNKS)
            def _():
                pltpu.async_copy(hp_hbm.at[_sidx(j + 1, 0)], nxt, nsem)

        for g in range(HK // 16):
            w16 = ew_v[pl.ds(j * K + h * HK + g * 16, 16)]
            for l in range(16):
                ws = w16.at[jnp.full((16,), l, jnp.int32)].get(
                    mode="promise_in_bounds")
                e = g * 16 + l
                for cg in range(8):
                    sl = pl.ds(cg * 16, 16)
                    cur[e, sl] = cur[e, sl] * ws
        pltpu.sync_copy(cur, acc.at[_didx(j, h)], add=True)

    pltpu.async_copy(hp_hbm.at[_sidx(0, 0)], rows_a, sem_a)

    def chunk(j, carry):
        section(j, 0, rows_a, rows_b, sem_a, sem_b, False)
        section(j, 1, rows_b, rows_a, sem_b, sem_a, False)
        return carry

    lax.fori_loop(0, CHUNKS - 1, chunk, 0)
    section(CHUNKS - 1, 0, rows_a, rows_b, sem_a, sem_b, False)
    section(CHUNKS - 1, 1, rows_b, rows_a, sem_b, sem_a, True)
    plsc.subcore_barrier()
    pltpu.sync_copy(acc.at[pl.ds(r0, RPT)], out_hbm.at[c, pl.ds(r0, RPT)])


def _tc1_body(deg0_ref, deg1_ref, x_ref, w_ref, dis_ref, hp_ref):
    deg = 1.0 + deg0_ref[...] + deg1_ref[...]
    dis = jnp.where(deg > 0, lax.rsqrt(deg), 0.0)
    dis_ref[...] = dis
    h = jnp.dot(x_ref[...], w_ref[...], preferred_element_type=jnp.float32,
                precision=lax.Precision.HIGHEST)
    hp_ref[...] = h * dis


def _tc_mid_body(a0_ref, a1_ref, hp_ref, dis_ref, b_ref, w_ref, out_ref):
    dis = dis_ref[...]
    t = (a0_ref[...] + a1_ref[...] + hp_ref[...]) * dis + b_ref[...]
    o = jnp.maximum(t, 0.0)
    out_ref[...] = jnp.dot(o, w_ref[...], preferred_element_type=jnp.float32,
                           precision=lax.Precision.HIGHEST) * dis


def _tc_fin_body(a0_ref, a1_ref, hp_ref, dis_ref, b_ref, brow_ref, wl_ref,
                 bl_ref, out_ref):
    o3 = (a0_ref[...] + a1_ref[...] + hp_ref[...]) * dis_ref[...] + b_ref[...]
    gid = lax.broadcasted_iota(jnp.int32, (G, N), 0)
    oh = (gid == brow_ref[...]).astype(jnp.float32)
    sums = jnp.dot(oh, o3, preferred_element_type=jnp.float32,
                   precision=lax.Precision.HIGHEST)
    cnt = jnp.dot(oh, jnp.ones((N, 1), jnp.float32),
                  preferred_element_type=jnp.float32,
                  precision=lax.Precision.HIGHEST)
    pooled = sums / jnp.maximum(cnt, 1.0)
    out_ref[...] = jnp.dot(pooled, wl_ref[...],
                           preferred_element_type=jnp.float32,
                           precision=lax.Precision.HIGHEST) + bl_ref[...]


def _pc(body, out_shapes):
    return pl.pallas_call(body, out_shape=out_shapes)


def _pad_to(a, n, dtype):
    return jnp.concatenate([a, jnp.zeros((n - a.shape[0],), dtype)])


def kernel(x, edge_index, edge_attr, batch, W1, b1, W2, b2, W3, b3, Wl, bl):
    src3 = _pad_to(edge_index[0], EPAD, jnp.int32).reshape(TILES, CHUNKS, K)
    dst3 = _pad_to(edge_index[1], EPAD, jnp.int32).reshape(TILES, CHUNKS, K)
    ew3 = _pad_to(edge_attr, EPAD, jnp.float32).reshape(TILES, CHUNKS, K)
    ew2 = ew3.reshape(TILES, CHUNKS * K)
    z1 = jnp.zeros((RPT,), jnp.float32)
    z2 = jnp.zeros((RPT, H), jnp.float32)
    brow = batch[None, :]

    deg2 = _deg_sc(dst3, ew3, z1)
    deg0 = deg2[0, :N, None]
    deg1 = deg2[1, :N, None]
    dis, hp1 = _pc(_tc1_body, [
        jax.ShapeDtypeStruct((N, 1), jnp.float32),
        jax.ShapeDtypeStruct((N, H), jnp.float32),
    ])(deg0, deg1, x, W1)

    a1 = _agg_sc(hp1, src3, dst3, ew2, z2)
    hp2 = _pc(_tc_mid_body, jax.ShapeDtypeStruct((N, H), jnp.float32))(
        a1[0, :N], a1[1, :N], hp1, dis, b1[None, :], W2)

    a2 = _agg_sc(hp2, src3, dst3, ew2, z2)
    hp3 = _pc(_tc_mid_body, jax.ShapeDtypeStruct((N, H), jnp.float32))(
        a2[0, :N], a2[1, :N], hp2, dis, b2[None, :], W3)

    a3 = _agg_sc(hp3, src3, dst3, ew2, z2)
    out = _pc(_tc_fin_body, jax.ShapeDtypeStruct((G, C), jnp.float32))(
        a3[0, :N], a3[1, :N], hp3, dis, b3[None, :], brow, Wl, bl[None, :])
    return out
